# initial kernel scaffold (unmeasured)
import jax
import jax.numpy as jnp
from jax import lax
from jax.experimental import pallas as pl
from jax.experimental.pallas import tpu as pltpu

N_DEV = 32
B, SQ, SKV, HQ_LOC, DH = 2, 128, 128, 4, 64
ROWS = B * SQ
CHUNK = ROWS // N_DEV
D_MODEL = 512
BH = B * HQ_LOC


def kernel(x, Wq, K_ext, V_ext, Wo):
    my = lax.axis_index("i")
    K = lax.dynamic_slice_in_dim(K_ext, my * HQ_LOC, HQ_LOC, axis=2)
    V = lax.dynamic_slice_in_dim(V_ext, my * HQ_LOC, HQ_LOC, axis=2)
    K = jnp.reshape(jnp.transpose(K, (0, 2, 1, 3)), (BH, SKV, DH))
    V = jnp.reshape(jnp.transpose(V, (0, 2, 1, 3)), (BH, SKV, DH))
    x2 = jnp.reshape(x, (ROWS, x.shape[-1]))

    def body(x_ref, wq_ref, k_ref, v_ref, wo_ref, out_ref,
             ctx_ref, acc_ref, red_ref, comm_ref,
             send1, recv1, send2, recv2):
        my_pos = lax.axis_index("i")

        xb = x_ref[...].astype(jnp.bfloat16)
        wq = wq_ref[...].astype(jnp.bfloat16)
        q2 = lax.dot_general(xb, wq, (((1,), (0,)), ((), ())),
                             preferred_element_type=jnp.float32)

        qb = lax.broadcasted_iota(jnp.int32, (SQ, SKV), 0) // 64
        kb = lax.broadcasted_iota(jnp.int32, (SQ, SKV), 1) // 64
        mask = (qb == kb) | (kb == 0) | (((qb + kb) % 3) == 0)

        for i in range(BH):
            b, h = i // HQ_LOC, i % HQ_LOC
            q = q2[b * SQ:(b + 1) * SQ, h * DH:(h + 1) * DH]
            k = k_ref[i]
            s = lax.dot_general(q.astype(jnp.bfloat16),
                                k.astype(jnp.bfloat16),
                                (((1,), (1,)), ((), ())),
                                preferred_element_type=jnp.float32) * 0.125
            s = jnp.where(mask, s, -1e9)
            m = jnp.max(s, axis=-1, keepdims=True)
            w = jnp.exp(s - m)
            w = w / jnp.sum(w, axis=-1, keepdims=True)
            ctx = lax.dot_general(w.astype(jnp.bfloat16),
                                  v_ref[i].astype(jnp.bfloat16),
                                  (((1,), (0,)), ((), ())),
                                  preferred_element_type=jnp.float32)
            ctx_ref[b * SQ:(b + 1) * SQ, h * DH:(h + 1) * DH] = ctx

        wo = wo_ref[...].astype(jnp.bfloat16)
        acc_ref[...] = lax.dot_general(ctx_ref[...].astype(jnp.bfloat16), wo,
                                       (((1,), (0,)), ((), ())),
                                       preferred_element_type=jnp.float32)

        for j in range(N_DEV):
            @pl.when(j != my_pos)
            def _():
                rdma = pltpu.make_async_remote_copy(
                    src_ref=acc_ref.at[pl.ds(j * CHUNK, CHUNK)],
                    dst_ref=comm_ref.at[my_pos],
                    send_sem=send1.at[j],
                    recv_sem=recv1.at[my_pos],
                    device_id=(j,),
                    device_id_type=pl.DeviceIdType.MESH,
                )
                rdma.start()

        for j in range(N_DEV):
            @pl.when(j != my_pos)
            def _():
                pltpu.make_async_remote_copy(
                    src_ref=acc_ref.at[pl.ds(0, CHUNK)],
                    dst_ref=comm_ref.at[j],
                    send_sem=send1.at[j],
                    recv_sem=recv1.at[j],
                    device_id=(0,),
                    device_id_type=pl.DeviceIdType.MESH,
                ).wait_recv()

        vals = comm_ref[...]
        slot = lax.broadcasted_iota(jnp.int32, vals.shape, 0)
        vals = jnp.where(slot == my_pos, 0.0, vals)
        own = acc_ref[pl.ds(my_pos * CHUNK, CHUNK), :]
        red = jnp.sum(vals, axis=0) + own
        red_ref[...] = red
        out_ref[pl.ds(my_pos * CHUNK, CHUNK), :] = red

        for j in range(N_DEV):
            @pl.when(j != my_pos)
            def _():
                rdma = pltpu.make_async_remote_copy(
                    src_ref=red_ref,
                    dst_ref=out_ref.at[pl.ds(my_pos * CHUNK, CHUNK)],
                    send_sem=send2.at[j],
                    recv_sem=recv2.at[my_pos],
                    device_id=(j,),
                    device_id_type=pl.DeviceIdType.MESH,
                )
                rdma.start()

        for j in range(N_DEV):
            @pl.when(j != my_pos)
            def _():
                pltpu.make_async_remote_copy(
                    src_ref=red_ref,
                    dst_ref=out_ref.at[pl.ds(j * CHUNK, CHUNK)],
                    send_sem=send2.at[j],
                    recv_sem=recv2.at[j],
                    device_id=(0,),
                    device_id_type=pl.DeviceIdType.MESH,
                ).wait_recv()

        for j in range(N_DEV):
            @pl.when(j != my_pos)
            def _():
                pltpu.make_async_remote_copy(
                    src_ref=acc_ref.at[pl.ds(j * CHUNK, CHUNK)],
                    dst_ref=comm_ref.at[j],
                    send_sem=send1.at[j],
                    recv_sem=recv1.at[j],
                    device_id=(0,),
                    device_id_type=pl.DeviceIdType.MESH,
                ).wait_send()
                pltpu.make_async_remote_copy(
                    src_ref=red_ref,
                    dst_ref=out_ref.at[pl.ds(j * CHUNK, CHUNK)],
                    send_sem=send2.at[j],
                    recv_sem=recv2.at[j],
                    device_id=(0,),
                    device_id_type=pl.DeviceIdType.MESH,
                ).wait_send()

    out = pl.pallas_call(
        body,
        out_shape=jax.ShapeDtypeStruct((ROWS, D_MODEL), jnp.float32),
        in_specs=[pl.BlockSpec(memory_space=pltpu.VMEM)] * 5,
        out_specs=pl.BlockSpec(memory_space=pltpu.VMEM),
        scratch_shapes=[
            pltpu.VMEM((ROWS, HQ_LOC * DH), jnp.float32),
            pltpu.VMEM((ROWS, D_MODEL), jnp.float32),
            pltpu.VMEM((CHUNK, D_MODEL), jnp.float32),
            pltpu.VMEM((N_DEV, CHUNK, D_MODEL), jnp.float32),
            pltpu.SemaphoreType.DMA((N_DEV,)),
            pltpu.SemaphoreType.DMA((N_DEV,)),
            pltpu.SemaphoreType.DMA((N_DEV,)),
            pltpu.SemaphoreType.DMA((N_DEV,)),
        ],
        compiler_params=pltpu.CompilerParams(collective_id=0),
    )(x2, Wq, K, V, Wo)
    return jnp.reshape(out, (B, SQ, D_MODEL))


# baseline (device time: 47039 ns/iter reference)
import jax
import jax.numpy as jnp
from jax import lax
from jax.experimental import pallas as pl
from jax.experimental.pallas import tpu as pltpu

N_DEV = 32
B, SQ, SKV, HQ_LOC, DH = 2, 128, 128, 4, 64
ROWS = B * SQ
CHUNK = ROWS // N_DEV
D_MODEL = 512
BH = B * HQ_LOC


def kernel(x, Wq, K_ext, V_ext, Wo):
    my = lax.axis_index("i")
    K = lax.dynamic_slice_in_dim(K_ext, my * HQ_LOC, HQ_LOC, axis=2)
    V = lax.dynamic_slice_in_dim(V_ext, my * HQ_LOC, HQ_LOC, axis=2)
    K = jnp.reshape(jnp.transpose(K, (0, 2, 1, 3)), (BH, SKV, DH))
    V = jnp.reshape(jnp.transpose(V, (0, 2, 1, 3)), (BH, SKV, DH))
    x2 = jnp.reshape(x, (ROWS, x.shape[-1]))

    def body(x_ref, wq_ref, k_ref, v_ref, wo_ref, out_ref,
             ctx_ref, acc_ref, red_ref, comm_ref,
             send1, recv1, send2, recv2):
        my_pos = lax.axis_index("i")

        xb = x_ref[...].astype(jnp.bfloat16)
        wq = wq_ref[...].astype(jnp.bfloat16)
        q2 = lax.dot_general(xb, wq, (((1,), (0,)), ((), ())),
                             preferred_element_type=jnp.float32)

        qb = lax.broadcasted_iota(jnp.int32, (SQ, SKV), 0) // 64
        kb = lax.broadcasted_iota(jnp.int32, (SQ, SKV), 1) // 64
        mask = (qb == kb) | (kb == 0) | (((qb + kb) % 3) == 0)

        for i in range(BH):
            b, h = i // HQ_LOC, i % HQ_LOC
            q = q2[b * SQ:(b + 1) * SQ, h * DH:(h + 1) * DH]
            k = k_ref[i]
            s = lax.dot_general(q.astype(jnp.bfloat16),
                                k.astype(jnp.bfloat16),
                                (((1,), (1,)), ((), ())),
                                preferred_element_type=jnp.float32) * 0.125
            s = jnp.where(mask, s, -1e9)
            m = jnp.max(s, axis=-1, keepdims=True)
            w = jnp.exp(s - m)
            w = w / jnp.sum(w, axis=-1, keepdims=True)
            ctx = lax.dot_general(w.astype(jnp.bfloat16),
                                  v_ref[i].astype(jnp.bfloat16),
                                  (((1,), (0,)), ((), ())),
                                  preferred_element_type=jnp.float32)
            ctx_ref[b * SQ:(b + 1) * SQ, h * DH:(h + 1) * DH] = ctx

        wo = wo_ref[...].astype(jnp.bfloat16)
        acc_ref[...] = lax.dot_general(ctx_ref[...].astype(jnp.bfloat16), wo,
                                       (((1,), (0,)), ((), ())),
                                       preferred_element_type=jnp.float32)

        for j in range(N_DEV):
            @pl.when(j != my_pos)
            def _():
                rdma = pltpu.make_async_remote_copy(
                    src_ref=acc_ref.at[pl.ds(j * CHUNK, CHUNK)],
                    dst_ref=comm_ref.at[my_pos],
                    send_sem=send1.at[j],
                    recv_sem=recv1.at[my_pos],
                    device_id=(j,),
                    device_id_type=pl.DeviceIdType.MESH,
                )
                rdma.start()

        for j in range(N_DEV):
            @pl.when(j != my_pos)
            def _():
                pltpu.make_async_remote_copy(
                    src_ref=acc_ref.at[pl.ds(0, CHUNK)],
                    dst_ref=comm_ref.at[j],
                    send_sem=send1.at[j],
                    recv_sem=recv1.at[j],
                    device_id=(0,),
                    device_id_type=pl.DeviceIdType.MESH,
                ).wait_recv()

        vals = comm_ref[...]
        slot = lax.broadcasted_iota(jnp.int32, vals.shape, 0)
        vals = jnp.where(slot == my_pos, 0.0, vals)
        own = acc_ref[pl.ds(my_pos * CHUNK, CHUNK), :]
        red = jnp.sum(vals, axis=0) + own
        red_ref[...] = red
        out_ref[pl.ds(my_pos * CHUNK, CHUNK), :] = red

        for j in range(N_DEV):
            @pl.when(j != my_pos)
            def _():
                rdma = pltpu.make_async_remote_copy(
                    src_ref=red_ref,
                    dst_ref=out_ref.at[pl.ds(my_pos * CHUNK, CHUNK)],
                    send_sem=send2.at[j],
                    recv_sem=recv2.at[my_pos],
                    device_id=(j,),
                    device_id_type=pl.DeviceIdType.MESH,
                )
                rdma.start()

        for j in range(N_DEV):
            @pl.when(j != my_pos)
            def _():
                pltpu.make_async_remote_copy(
                    src_ref=red_ref,
                    dst_ref=out_ref.at[pl.ds(j * CHUNK, CHUNK)],
                    send_sem=send2.at[j],
                    recv_sem=recv2.at[j],
                    device_id=(0,),
                    device_id_type=pl.DeviceIdType.MESH,
                ).wait_recv()

        for j in range(N_DEV):
            @pl.when(j != my_pos)
            def _():
                pltpu.make_async_remote_copy(
                    src_ref=acc_ref.at[pl.ds(j * CHUNK, CHUNK)],
                    dst_ref=comm_ref.at[j],
                    send_sem=send1.at[j],
                    recv_sem=recv1.at[j],
                    device_id=(0,),
                    device_id_type=pl.DeviceIdType.MESH,
                ).wait_send()
                pltpu.make_async_remote_copy(
                    src_ref=red_ref,
                    dst_ref=out_ref.at[pl.ds(j * CHUNK, CHUNK)],
                    send_sem=send2.at[j],
                    recv_sem=recv2.at[j],
                    device_id=(0,),
                    device_id_type=pl.DeviceIdType.MESH,
                ).wait_send()

    out = pl.pallas_call(
        body,
        out_shape=jax.ShapeDtypeStruct((ROWS, D_MODEL), jnp.float32),
        in_specs=[pl.BlockSpec(memory_space=pltpu.VMEM)] * 5,
        out_specs=pl.BlockSpec(memory_space=pltpu.VMEM),
        scratch_shapes=[
            pltpu.VMEM((ROWS, HQ_LOC * DH), jnp.float32),
            pltpu.VMEM((ROWS, D_MODEL), jnp.float32),
            pltpu.VMEM((CHUNK, D_MODEL), jnp.float32),
            pltpu.VMEM((N_DEV, CHUNK, D_MODEL), jnp.float32),
            pltpu.SemaphoreType.DMA((N_DEV,)),
            pltpu.SemaphoreType.DMA((N_DEV,)),
            pltpu.SemaphoreType.DMA((N_DEV,)),
            pltpu.SemaphoreType.DMA((N_DEV,)),
        ],
    )(x2, Wq, K, V, Wo)
    return jnp.reshape(out, (B, SQ, D_MODEL))


# device time: 46957 ns/iter; 1.0017x vs baseline; 1.0017x over previous
import jax
import jax.numpy as jnp
from jax import lax
from jax.experimental import pallas as pl
from jax.experimental.pallas import tpu as pltpu

N_DEV = 32
B, SQ, SKV, HQ_LOC, DH = 2, 128, 128, 4, 64
ROWS = B * SQ
CHUNK = ROWS // N_DEV
D_MODEL = 512


def kernel(x, Wq, K_ext, V_ext, Wo):
    def body(x_ref, wq_ref, k_hbm, v_hbm, wo_ref, out_ref,
             k_ref, v_ref, ctx_ref, acc_ref, accb_ref, red_ref,
             comm_ref, gat_ref,
             kv_sems, send1, recv1, send2, recv2):
        my_pos = lax.axis_index("i")

        kdma = pltpu.make_async_copy(
            k_hbm.at[:, :, pl.ds(my_pos * HQ_LOC, HQ_LOC), :], k_ref,
            kv_sems.at[0])
        vdma = pltpu.make_async_copy(
            v_hbm.at[:, :, pl.ds(my_pos * HQ_LOC, HQ_LOC), :], v_ref,
            kv_sems.at[1])
        kdma.start()
        vdma.start()

        xb = jnp.reshape(x_ref[...], (ROWS, D_MODEL)).astype(jnp.bfloat16)
        wq = wq_ref[...].astype(jnp.bfloat16)
        q2 = lax.dot_general(xb, wq, (((1,), (0,)), ((), ())),
                             preferred_element_type=jnp.float32)

        qb = lax.broadcasted_iota(jnp.int32, (SQ, SKV), 0) // 64
        kb = lax.broadcasted_iota(jnp.int32, (SQ, SKV), 1) // 64
        mask = (qb == kb) | (kb == 0) | (((qb + kb) % 3) == 0)

        kdma.wait()
        vdma.wait()
        for b in range(B):
            for h in range(HQ_LOC):
                q = q2[b * SQ:(b + 1) * SQ, h * DH:(h + 1) * DH]
                k = k_ref[b, :, h, :]
                s = lax.dot_general(q.astype(jnp.bfloat16),
                                    k.astype(jnp.bfloat16),
                                    (((1,), (1,)), ((), ())),
                                    preferred_element_type=jnp.float32) * 0.125
                s = jnp.where(mask, s, -1e9)
                m = jnp.max(s, axis=-1, keepdims=True)
                w = jnp.exp(s - m)
                w = w / jnp.sum(w, axis=-1, keepdims=True)
                ctx = lax.dot_general(w.astype(jnp.bfloat16),
                                      v_ref[b, :, h, :].astype(jnp.bfloat16),
                                      (((1,), (0,)), ((), ())),
                                      preferred_element_type=jnp.float32)
                ctx_ref[b * SQ:(b + 1) * SQ, h * DH:(h + 1) * DH] = ctx

        wo = wo_ref[...].astype(jnp.bfloat16)
        acc = lax.dot_general(ctx_ref[...].astype(jnp.bfloat16), wo,
                              (((1,), (0,)), ((), ())),
                              preferred_element_type=jnp.float32)
        acc_ref[...] = acc
        accb_ref[...] = acc.astype(jnp.bfloat16)

        for j in range(N_DEV):
            @pl.when(j != my_pos)
            def _():
                pltpu.make_async_remote_copy(
                    src_ref=accb_ref.at[pl.ds(j * CHUNK, CHUNK)],
                    dst_ref=comm_ref.at[my_pos],
                    send_sem=send1.at[j],
                    recv_sem=recv1.at[my_pos],
                    device_id=(j,),
                    device_id_type=pl.DeviceIdType.MESH,
                ).start()

        for j in range(N_DEV):
            @pl.when(j != my_pos)
            def _():
                pltpu.make_async_remote_copy(
                    src_ref=accb_ref.at[pl.ds(0, CHUNK)],
                    dst_ref=comm_ref.at[j],
                    send_sem=send1.at[j],
                    recv_sem=recv1.at[j],
                    device_id=(0,),
                    device_id_type=pl.DeviceIdType.MESH,
                ).wait_recv()

        vals = comm_ref[...].astype(jnp.float32)
        slot = lax.broadcasted_iota(jnp.int32, vals.shape, 0)
        vals = jnp.where(slot == my_pos, 0.0, vals)
        own = acc_ref[pl.ds(my_pos * CHUNK, CHUNK), :]
        red = jnp.sum(vals, axis=0) + own
        red_ref[...] = red.astype(jnp.bfloat16)

        for j in range(N_DEV):
            @pl.when(j != my_pos)
            def _():
                pltpu.make_async_remote_copy(
                    src_ref=red_ref,
                    dst_ref=gat_ref.at[my_pos],
                    send_sem=send2.at[j],
                    recv_sem=recv2.at[my_pos],
                    device_id=(j,),
                    device_id_type=pl.DeviceIdType.MESH,
                ).start()

        for j in range(N_DEV):
            @pl.when(j != my_pos)
            def _():
                pltpu.make_async_remote_copy(
                    src_ref=red_ref,
                    dst_ref=gat_ref.at[j],
                    send_sem=send2.at[j],
                    recv_sem=recv2.at[j],
                    device_id=(0,),
                    device_id_type=pl.DeviceIdType.MESH,
                ).wait_recv()

        out_ref[...] = jnp.reshape(gat_ref[...],
                                   (ROWS, D_MODEL)).astype(jnp.float32)
        out_ref[pl.ds(my_pos * CHUNK, CHUNK), :] = red

        for j in range(N_DEV):
            @pl.when(j != my_pos)
            def _():
                pltpu.make_async_remote_copy(
                    src_ref=accb_ref.at[pl.ds(j * CHUNK, CHUNK)],
                    dst_ref=comm_ref.at[j],
                    send_sem=send1.at[j],
                    recv_sem=recv1.at[j],
                    device_id=(0,),
                    device_id_type=pl.DeviceIdType.MESH,
                ).wait_send()
                pltpu.make_async_remote_copy(
                    src_ref=red_ref,
                    dst_ref=gat_ref.at[j],
                    send_sem=send2.at[j],
                    recv_sem=recv2.at[j],
                    device_id=(0,),
                    device_id_type=pl.DeviceIdType.MESH,
                ).wait_send()

    out = pl.pallas_call(
        body,
        out_shape=jax.ShapeDtypeStruct((ROWS, D_MODEL), jnp.float32),
        in_specs=[
            pl.BlockSpec(memory_space=pltpu.VMEM),
            pl.BlockSpec(memory_space=pltpu.VMEM),
            pl.BlockSpec(memory_space=pl.ANY),
            pl.BlockSpec(memory_space=pl.ANY),
            pl.BlockSpec(memory_space=pltpu.VMEM),
        ],
        out_specs=pl.BlockSpec(memory_space=pltpu.VMEM),
        scratch_shapes=[
            pltpu.VMEM((B, SKV, HQ_LOC, DH), jnp.float32),
            pltpu.VMEM((B, SKV, HQ_LOC, DH), jnp.float32),
            pltpu.VMEM((ROWS, HQ_LOC * DH), jnp.float32),
            pltpu.VMEM((ROWS, D_MODEL), jnp.float32),
            pltpu.VMEM((ROWS, D_MODEL), jnp.bfloat16),
            pltpu.VMEM((CHUNK, D_MODEL), jnp.bfloat16),
            pltpu.VMEM((N_DEV, CHUNK, D_MODEL), jnp.bfloat16),
            pltpu.VMEM((N_DEV, CHUNK, D_MODEL), jnp.bfloat16),
            pltpu.SemaphoreType.DMA((2,)),
            pltpu.SemaphoreType.DMA((N_DEV,)),
            pltpu.SemaphoreType.DMA((N_DEV,)),
            pltpu.SemaphoreType.DMA((N_DEV,)),
            pltpu.SemaphoreType.DMA((N_DEV,)),
        ],
    )(x, Wq, K_ext, V_ext, Wo)
    return jnp.reshape(out, (B, SQ, D_MODEL))


# device time: 38507 ns/iter; 1.2216x vs baseline; 1.2194x over previous
import jax
import jax.numpy as jnp
from jax import lax
from jax.experimental import pallas as pl
from jax.experimental.pallas import tpu as pltpu

N_DEV = 32
B, SQ, SKV, HQ_LOC, DH = 2, 128, 128, 4, 64
ROWS = B * SQ
CHUNK = ROWS // N_DEV
D_MODEL = 512


def kernel(x, Wq, K_ext, V_ext, Wo):
    def body(x_ref, wq_ref, k_hbm, v_hbm, wo_ref, out_ref,
             k_ref, v_ref, ctx_ref, acc_ref, accb_ref, red_ref,
             comm_ref, gat_ref,
             kv_sems, send1, recv1, send2, recv2):
        my_pos = lax.axis_index("i")

        bar = pltpu.get_barrier_semaphore()
        for nbr_off in (1, N_DEV - 1):
            pl.semaphore_signal(bar, inc=1,
                                device_id=((my_pos + nbr_off) % N_DEV,),
                                device_id_type=pl.DeviceIdType.MESH)
        pl.semaphore_wait(bar, 2)

        kdma = pltpu.make_async_copy(
            k_hbm.at[:, :, pl.ds(my_pos * HQ_LOC, HQ_LOC), :], k_ref,
            kv_sems.at[0])
        vdma = pltpu.make_async_copy(
            v_hbm.at[:, :, pl.ds(my_pos * HQ_LOC, HQ_LOC), :], v_ref,
            kv_sems.at[1])
        kdma.start()
        vdma.start()

        xb = jnp.reshape(x_ref[...], (ROWS, D_MODEL)).astype(jnp.bfloat16)
        wq = wq_ref[...].astype(jnp.bfloat16)
        q2 = lax.dot_general(xb, wq, (((1,), (0,)), ((), ())),
                             preferred_element_type=jnp.float32)

        qb = lax.broadcasted_iota(jnp.int32, (SQ, SKV), 0) // 64
        kb = lax.broadcasted_iota(jnp.int32, (SQ, SKV), 1) // 64
        mask = (qb == kb) | (kb == 0) | (((qb + kb) % 3) == 0)

        kdma.wait()
        vdma.wait()
        for b in range(B):
            for h in range(HQ_LOC):
                q = q2[b * SQ:(b + 1) * SQ, h * DH:(h + 1) * DH]
                k = k_ref[b, :, h, :]
                s = lax.dot_general(q.astype(jnp.bfloat16),
                                    k.astype(jnp.bfloat16),
                                    (((1,), (1,)), ((), ())),
                                    preferred_element_type=jnp.float32) * 0.125
                s = jnp.where(mask, s, -1e9)
                m = jnp.max(s, axis=-1, keepdims=True)
                w = jnp.exp(s - m)
                w = w / jnp.sum(w, axis=-1, keepdims=True)
                ctx = lax.dot_general(w.astype(jnp.bfloat16),
                                      v_ref[b, :, h, :].astype(jnp.bfloat16),
                                      (((1,), (0,)), ((), ())),
                                      preferred_element_type=jnp.float32)
                ctx_ref[b * SQ:(b + 1) * SQ, h * DH:(h + 1) * DH] = ctx

        wo = wo_ref[...].astype(jnp.bfloat16)
        acc = lax.dot_general(ctx_ref[...].astype(jnp.bfloat16), wo,
                              (((1,), (0,)), ((), ())),
                              preferred_element_type=jnp.float32)
        acc_ref[...] = acc
        accb_ref[...] = acc.astype(jnp.bfloat16)

        for j in range(N_DEV):
            @pl.when(j != my_pos)
            def _():
                pltpu.make_async_remote_copy(
                    src_ref=accb_ref.at[pl.ds(j * CHUNK, CHUNK)],
                    dst_ref=comm_ref.at[my_pos],
                    send_sem=send1.at[j],
                    recv_sem=recv1.at[my_pos],
                    device_id=(j,),
                    device_id_type=pl.DeviceIdType.MESH,
                ).start()

        for j in range(N_DEV):
            @pl.when(j != my_pos)
            def _():
                pltpu.make_async_remote_copy(
                    src_ref=accb_ref.at[pl.ds(0, CHUNK)],
                    dst_ref=comm_ref.at[j],
                    send_sem=send1.at[j],
                    recv_sem=recv1.at[j],
                    device_id=(0,),
                    device_id_type=pl.DeviceIdType.MESH,
                ).wait_recv()

        vals = comm_ref[...].astype(jnp.float32)
        slot = lax.broadcasted_iota(jnp.int32, vals.shape, 0)
        vals = jnp.where(slot == my_pos, 0.0, vals)
        own = acc_ref[pl.ds(my_pos * CHUNK, CHUNK), :]
        red = jnp.sum(vals, axis=0) + own
        red_ref[...] = red.astype(jnp.bfloat16)

        for j in range(N_DEV):
            @pl.when(j != my_pos)
            def _():
                pltpu.make_async_remote_copy(
                    src_ref=red_ref,
                    dst_ref=gat_ref.at[my_pos],
                    send_sem=send2.at[j],
                    recv_sem=recv2.at[my_pos],
                    device_id=(j,),
                    device_id_type=pl.DeviceIdType.MESH,
                ).start()

        for j in range(N_DEV):
            @pl.when(j != my_pos)
            def _():
                pltpu.make_async_remote_copy(
                    src_ref=red_ref,
                    dst_ref=gat_ref.at[j],
                    send_sem=send2.at[j],
                    recv_sem=recv2.at[j],
                    device_id=(0,),
                    device_id_type=pl.DeviceIdType.MESH,
                ).wait_recv()

        out_ref[...] = jnp.reshape(gat_ref[...],
                                   (ROWS, D_MODEL)).astype(jnp.float32)
        out_ref[pl.ds(my_pos * CHUNK, CHUNK), :] = red

        for j in range(N_DEV):
            @pl.when(j != my_pos)
            def _():
                pltpu.make_async_remote_copy(
                    src_ref=accb_ref.at[pl.ds(j * CHUNK, CHUNK)],
                    dst_ref=comm_ref.at[j],
                    send_sem=send1.at[j],
                    recv_sem=recv1.at[j],
                    device_id=(0,),
                    device_id_type=pl.DeviceIdType.MESH,
                ).wait_send()
                pltpu.make_async_remote_copy(
                    src_ref=red_ref,
                    dst_ref=gat_ref.at[j],
                    send_sem=send2.at[j],
                    recv_sem=recv2.at[j],
                    device_id=(0,),
                    device_id_type=pl.DeviceIdType.MESH,
                ).wait_send()

    out = pl.pallas_call(
        body,
        out_shape=jax.ShapeDtypeStruct((ROWS, D_MODEL), jnp.float32),
        in_specs=[
            pl.BlockSpec(memory_space=pltpu.VMEM),
            pl.BlockSpec(memory_space=pltpu.VMEM),
            pl.BlockSpec(memory_space=pl.ANY),
            pl.BlockSpec(memory_space=pl.ANY),
            pl.BlockSpec(memory_space=pltpu.VMEM),
        ],
        out_specs=pl.BlockSpec(memory_space=pltpu.VMEM),
        scratch_shapes=[
            pltpu.VMEM((B, SKV, HQ_LOC, DH), jnp.float32),
            pltpu.VMEM((B, SKV, HQ_LOC, DH), jnp.float32),
            pltpu.VMEM((ROWS, HQ_LOC * DH), jnp.float32),
            pltpu.VMEM((ROWS, D_MODEL), jnp.float32),
            pltpu.VMEM((ROWS, D_MODEL), jnp.bfloat16),
            pltpu.VMEM((CHUNK, D_MODEL), jnp.bfloat16),
            pltpu.VMEM((N_DEV, CHUNK, D_MODEL), jnp.bfloat16),
            pltpu.VMEM((N_DEV, CHUNK, D_MODEL), jnp.bfloat16),
            pltpu.SemaphoreType.DMA((2,)),
            pltpu.SemaphoreType.DMA((N_DEV,)),
            pltpu.SemaphoreType.DMA((N_DEV,)),
            pltpu.SemaphoreType.DMA((N_DEV,)),
            pltpu.SemaphoreType.DMA((N_DEV,)),
        ],
        compiler_params=pltpu.CompilerParams(collective_id=0),
    )(x, Wq, K_ext, V_ext, Wo)
    return jnp.reshape(out, (B, SQ, D_MODEL))


# device time: 38502 ns/iter; 1.2217x vs baseline; 1.0001x over previous
import jax
import jax.numpy as jnp
from jax import lax
from jax.experimental import pallas as pl
from jax.experimental.pallas import tpu as pltpu

N_DEV = 32
B, SQ, SKV, HQ_LOC, DH = 2, 128, 128, 4, 64
ROWS = B * SQ
CHUNK = ROWS // N_DEV
D_MODEL = 512


def kernel(x, Wq, K_ext, V_ext, Wo):
    def body(x_hbm, wq_hbm, k_hbm, v_hbm, wo_hbm, out_ref,
             x_ref, wq_ref, wo_ref,
             k_ref, v_ref, ctx_ref, acc_ref, accb_ref, red_ref,
             comm_ref, gat_ref,
             in_sems, kv_sems, send1, recv1, send2, recv2):
        my_pos = lax.axis_index("i")

        xdma = pltpu.make_async_copy(x_hbm, x_ref, in_sems.at[0])
        wqdma = pltpu.make_async_copy(wq_hbm, wq_ref, in_sems.at[1])
        wodma = pltpu.make_async_copy(wo_hbm, wo_ref, in_sems.at[2])
        kdma = pltpu.make_async_copy(
            k_hbm.at[:, :, pl.ds(my_pos * HQ_LOC, HQ_LOC), :], k_ref,
            kv_sems.at[0])
        vdma = pltpu.make_async_copy(
            v_hbm.at[:, :, pl.ds(my_pos * HQ_LOC, HQ_LOC), :], v_ref,
            kv_sems.at[1])
        xdma.start()
        wqdma.start()
        wodma.start()
        kdma.start()
        vdma.start()

        bar = pltpu.get_barrier_semaphore()
        for nbr_off in (1, N_DEV - 1):
            pl.semaphore_signal(bar, inc=1,
                                device_id=((my_pos + nbr_off) % N_DEV,),
                                device_id_type=pl.DeviceIdType.MESH)
        pl.semaphore_wait(bar, 2)

        xdma.wait()
        wqdma.wait()
        xb = jnp.reshape(x_ref[...], (ROWS, D_MODEL)).astype(jnp.bfloat16)
        wq = wq_ref[...].astype(jnp.bfloat16)
        q2 = lax.dot_general(xb, wq, (((1,), (0,)), ((), ())),
                             preferred_element_type=jnp.float32)

        qb = lax.broadcasted_iota(jnp.int32, (SQ, SKV), 0) // 64
        kb = lax.broadcasted_iota(jnp.int32, (SQ, SKV), 1) // 64
        mask = (qb == kb) | (kb == 0) | (((qb + kb) % 3) == 0)

        kdma.wait()
        vdma.wait()
        for b in range(B):
            for h in range(HQ_LOC):
                q = q2[b * SQ:(b + 1) * SQ, h * DH:(h + 1) * DH]
                k = k_ref[b, :, h, :]
                s = lax.dot_general(q.astype(jnp.bfloat16),
                                    k.astype(jnp.bfloat16),
                                    (((1,), (1,)), ((), ())),
                                    preferred_element_type=jnp.float32) * 0.125
                s = jnp.where(mask, s, -1e9)
                m = jnp.max(s, axis=-1, keepdims=True)
                w = jnp.exp(s - m)
                w = w / jnp.sum(w, axis=-1, keepdims=True)
                ctx = lax.dot_general(w.astype(jnp.bfloat16),
                                      v_ref[b, :, h, :].astype(jnp.bfloat16),
                                      (((1,), (0,)), ((), ())),
                                      preferred_element_type=jnp.float32)
                ctx_ref[b * SQ:(b + 1) * SQ, h * DH:(h + 1) * DH] = ctx

        wodma.wait()
        wo = wo_ref[...].astype(jnp.bfloat16)
        acc = lax.dot_general(ctx_ref[...].astype(jnp.bfloat16), wo,
                              (((1,), (0,)), ((), ())),
                              preferred_element_type=jnp.float32)
        acc_ref[...] = acc
        accb_ref[...] = acc.astype(jnp.bfloat16)

        for j in range(N_DEV):
            @pl.when(j != my_pos)
            def _():
                pltpu.make_async_remote_copy(
                    src_ref=accb_ref.at[pl.ds(j * CHUNK, CHUNK)],
                    dst_ref=comm_ref.at[my_pos],
                    send_sem=send1.at[j],
                    recv_sem=recv1.at[my_pos],
                    device_id=(j,),
                    device_id_type=pl.DeviceIdType.MESH,
                ).start()

        for j in range(N_DEV):
            @pl.when(j != my_pos)
            def _():
                pltpu.make_async_remote_copy(
                    src_ref=accb_ref.at[pl.ds(0, CHUNK)],
                    dst_ref=comm_ref.at[j],
                    send_sem=send1.at[j],
                    recv_sem=recv1.at[j],
                    device_id=(0,),
                    device_id_type=pl.DeviceIdType.MESH,
                ).wait_recv()

        vals = comm_ref[...].astype(jnp.float32)
        slot = lax.broadcasted_iota(jnp.int32, vals.shape, 0)
        vals = jnp.where(slot == my_pos, 0.0, vals)
        own = acc_ref[pl.ds(my_pos * CHUNK, CHUNK), :]
        red = jnp.sum(vals, axis=0) + own
        red_ref[...] = red.astype(jnp.bfloat16)

        for j in range(N_DEV):
            @pl.when(j != my_pos)
            def _():
                pltpu.make_async_remote_copy(
                    src_ref=red_ref,
                    dst_ref=gat_ref.at[my_pos],
                    send_sem=send2.at[j],
                    recv_sem=recv2.at[my_pos],
                    device_id=(j,),
                    device_id_type=pl.DeviceIdType.MESH,
                ).start()

        for j in range(N_DEV):
            @pl.when(j != my_pos)
            def _():
                pltpu.make_async_remote_copy(
                    src_ref=red_ref,
                    dst_ref=gat_ref.at[j],
                    send_sem=send2.at[j],
                    recv_sem=recv2.at[j],
                    device_id=(0,),
                    device_id_type=pl.DeviceIdType.MESH,
                ).wait_recv()

        out_ref[...] = jnp.reshape(gat_ref[...],
                                   (ROWS, D_MODEL)).astype(jnp.float32)
        out_ref[pl.ds(my_pos * CHUNK, CHUNK), :] = red

        for j in range(N_DEV):
            @pl.when(j != my_pos)
            def _():
                pltpu.make_async_remote_copy(
                    src_ref=accb_ref.at[pl.ds(j * CHUNK, CHUNK)],
                    dst_ref=comm_ref.at[j],
                    send_sem=send1.at[j],
                    recv_sem=recv1.at[j],
                    device_id=(0,),
                    device_id_type=pl.DeviceIdType.MESH,
                ).wait_send()
                pltpu.make_async_remote_copy(
                    src_ref=red_ref,
                    dst_ref=gat_ref.at[j],
                    send_sem=send2.at[j],
                    recv_sem=recv2.at[j],
                    device_id=(0,),
                    device_id_type=pl.DeviceIdType.MESH,
                ).wait_send()

    out = pl.pallas_call(
        body,
        out_shape=jax.ShapeDtypeStruct((ROWS, D_MODEL), jnp.float32),
        in_specs=[pl.BlockSpec(memory_space=pl.ANY)] * 5,
        out_specs=pl.BlockSpec(memory_space=pltpu.VMEM),
        scratch_shapes=[
            pltpu.VMEM((B, SQ, D_MODEL), jnp.float32),
            pltpu.VMEM((D_MODEL, HQ_LOC * DH), jnp.float32),
            pltpu.VMEM((HQ_LOC * DH, D_MODEL), jnp.float32),
            pltpu.VMEM((B, SKV, HQ_LOC, DH), jnp.float32),
            pltpu.VMEM((B, SKV, HQ_LOC, DH), jnp.float32),
            pltpu.VMEM((ROWS, HQ_LOC * DH), jnp.float32),
            pltpu.VMEM((ROWS, D_MODEL), jnp.float32),
            pltpu.VMEM((ROWS, D_MODEL), jnp.bfloat16),
            pltpu.VMEM((CHUNK, D_MODEL), jnp.bfloat16),
            pltpu.VMEM((N_DEV, CHUNK, D_MODEL), jnp.bfloat16),
            pltpu.VMEM((N_DEV, CHUNK, D_MODEL), jnp.bfloat16),
            pltpu.SemaphoreType.DMA((3,)),
            pltpu.SemaphoreType.DMA((2,)),
            pltpu.SemaphoreType.DMA((N_DEV,)),
            pltpu.SemaphoreType.DMA((N_DEV,)),
            pltpu.SemaphoreType.DMA((N_DEV,)),
            pltpu.SemaphoreType.DMA((N_DEV,)),
        ],
        compiler_params=pltpu.CompilerParams(collective_id=0),
    )(x, Wq, K_ext, V_ext, Wo)
    return jnp.reshape(out, (B, SQ, D_MODEL))
